# Initial kernel scaffold; baseline (speedup 1.0000x reference)
import jax
import jax.numpy as jnp
from jax.experimental import pallas as pl


def kernel(users, pos, neg, table, s):
    raise NotImplementedError("write your pallas kernel here")


# speedup vs baseline: 1.0969x; 1.0969x over previous
import functools

import jax
import jax.numpy as jnp
from jax import lax
from jax.experimental import pallas as pl
from jax.experimental.pallas import tpu as pltpu
from jax.experimental.pallas import tpu_sc as plsc

B = 16384
D = 64
NC = 2
NS = 16
NW = NC * NS
BPW = B // NW
NCH = 4
CH = BPW // NCH
NG = BPW // 16
REG_W = 1e-4
SREG_W = 1e-3

_mesh = plsc.VectorSubcoreMesh(
    core_axis_name="c", subcore_axis_name="s", num_cores=NC, num_subcores=NS
)


@functools.partial(
    pl.kernel,
    out_type=[
        jax.ShapeDtypeStruct((B,), jnp.float32),
        jax.ShapeDtypeStruct((NW, 16), jnp.float32),
    ],
    mesh=_mesh,
    compiler_params=pltpu.CompilerParams(use_tc_tiling_on_sc=False),
    scratch_types=[
        pltpu.VMEM((NCH, CH), jnp.int32),
        pltpu.VMEM((NCH, CH), jnp.int32),
        pltpu.VMEM((NCH, CH), jnp.int32),
        pltpu.VMEM((BPW, D), jnp.float32),
        pltpu.VMEM((BPW, D), jnp.float32),
        pltpu.VMEM((BPW, D), jnp.float32),
        pltpu.VMEM((BPW,), jnp.float32),
        pltpu.VMEM((16,), jnp.float32),
        pltpu.VMEM((16,), jnp.float32),
        pltpu.SemaphoreType.DMA,
    ],
)
def _sc_gather(u_idx, p_idx, n_idx, thr, table, diffs_out, regs_out,
               iu, ip_, in_, ru, rp, rn, dbuf, thrv, rstage, sem):
    wid = lax.axis_index("s") * NC + lax.axis_index("c")
    ibase = wid * NCH

    pltpu.sync_copy(u_idx.at[pl.ds(ibase, NCH)], iu)
    pltpu.sync_copy(p_idx.at[pl.ds(ibase, NCH)], ip_)
    pltpu.sync_copy(n_idx.at[pl.ds(ibase, NCH)], in_)
    pltpu.sync_copy(thr, thrv)

    copies = []
    for j in range(NCH):
        dst = pl.ds(j * CH, CH)
        copies.append(pltpu.async_copy(table.at[iu.at[j]], ru.at[dst], sem))
        copies.append(pltpu.async_copy(table.at[ip_.at[j]], rp.at[dst], sem))
        copies.append(pltpu.async_copy(table.at[in_.at[j]], rn.at[dst], sem))
    for c in copies:
        c.wait()

    t = thrv[...]
    nt = -t
    lanes = lax.iota(jnp.int32, 16)

    def lane_sum(x):
        for sft in (8, 4, 2, 1):
            x = x + x.at[lanes ^ sft].get(mode="promise_in_bounds")
        return x

    def body(g, racc):
        rbase = g * 16
        dv = jnp.zeros((16,), jnp.float32)
        for r in range(16):
            row = rbase + r
            acc = jnp.zeros((16,), jnp.float32)
            for k in range(D // 16):
                sl = pl.ds(k * 16, 16)
                u = ru[row, sl]
                p = rp[row, sl]
                n = rn[row, sl]
                su = u - jnp.minimum(jnp.maximum(u, nt), t)
                sp = p - jnp.minimum(jnp.maximum(p, nt), t)
                sn = n - jnp.minimum(jnp.maximum(n, nt), t)
                acc = acc + su * (sn - sp)
                racc = racc + (u * u + p * p + n * n)
            dv = jnp.where(lanes == r, lane_sum(acc), dv)
        dbuf[pl.ds(rbase, 16)] = dv
        return racc

    racc = lax.fori_loop(0, NG, body, jnp.zeros((16,), jnp.float32))

    rstage[...] = racc
    pltpu.sync_copy(dbuf, diffs_out.at[pl.ds(wid * BPW, BPW)])
    pltpu.sync_copy(rstage, regs_out.at[wid])


def _tc_body(d_ref, r_ref, s_ref, loss_ref, le_ref, reg_ref, sl_ref):
    diff = d_ref[...]
    le = jnp.mean(jax.nn.softplus(diff))
    reg = 0.5 * jnp.sum(r_ref[...]) * (1.0 / B) * REG_W
    sv = s_ref[0]
    sl = 0.5 * sv * sv * (1.0 / B) * SREG_W
    le_ref[0] = le
    reg_ref[0] = reg
    sl_ref[0] = sl
    loss_ref[0] = le + reg + sl


_tc_final = pl.pallas_call(
    _tc_body,
    out_shape=[jax.ShapeDtypeStruct((1,), jnp.float32)] * 4,
    in_specs=[
        pl.BlockSpec(memory_space=pltpu.VMEM),
        pl.BlockSpec(memory_space=pltpu.VMEM),
        pl.BlockSpec(memory_space=pltpu.SMEM),
    ],
    out_specs=[pl.BlockSpec(memory_space=pltpu.SMEM)] * 4,
)


def kernel(users, pos, neg, table, s):
    s = s.astype(jnp.float32)
    u2 = users.astype(jnp.int32).reshape(B // CH, CH)
    p2 = pos.astype(jnp.int32).reshape(B // CH, CH)
    n2 = neg.astype(jnp.int32).reshape(B // CH, CH)
    thr16 = jnp.broadcast_to(jax.nn.sigmoid(s), (16,))
    diffs, regs = _sc_gather(u2, p2, n2, thr16, table)
    loss, le, reg, sl = _tc_final(diffs.reshape(B // CH, CH), regs, s)
    return loss[0], le[0], reg[0], sl[0]


# speedup vs baseline: 1.8198x; 1.6590x over previous
import functools

import jax
import jax.numpy as jnp
from jax import lax
from jax.experimental import pallas as pl
from jax.experimental.pallas import tpu as pltpu
from jax.experimental.pallas import tpu_sc as plsc

B = 16384
D = 64
NC = 2
NS = 16
NW = NC * NS
BPW = B // NW
NCH = 4
CH = BPW // NCH
NPASS = 4
PCH = BPW // NPASS
REG_W = 1e-4
SREG_W = 1e-3

_mesh = plsc.VectorSubcoreMesh(
    core_axis_name="c", subcore_axis_name="s", num_cores=NC, num_subcores=NS
)


@functools.partial(
    pl.kernel,
    out_type=[
        jax.ShapeDtypeStruct((B,), jnp.float32),
        jax.ShapeDtypeStruct((NW, 16), jnp.float32),
    ],
    mesh=_mesh,
    scratch_types=[
        pltpu.VMEM((NCH, CH), jnp.int32),
        pltpu.VMEM((NCH, CH), jnp.int32),
        pltpu.VMEM((NCH, CH), jnp.int32),
        pltpu.VMEM((PCH, D), jnp.float32),
        pltpu.VMEM((PCH, D), jnp.float32),
        pltpu.VMEM((PCH, D), jnp.float32),
        pltpu.VMEM((BPW,), jnp.float32),
        pltpu.VMEM((16,), jnp.float32),
        pltpu.VMEM((16,), jnp.float32),
        pltpu.SemaphoreType.DMA,
    ],
)
def _sc_gather(u_idx, p_idx, n_idx, thr, table, diffs_out, regs_out,
               iu, ip_, in_, ru, rp, rn, dbuf, thrv, rstage, sem):
    wid = lax.axis_index("s") * NC + lax.axis_index("c")
    ibase = wid * NCH

    pltpu.sync_copy(u_idx.at[pl.ds(ibase, NCH)], iu)
    pltpu.sync_copy(p_idx.at[pl.ds(ibase, NCH)], ip_)
    pltpu.sync_copy(n_idx.at[pl.ds(ibase, NCH)], in_)
    pltpu.sync_copy(thr, thrv)

    t = thrv[...]
    nt = -t
    lanes = lax.iota(jnp.int32, 16)

    def lane_sum(x):
        for sft in (8, 4, 2, 1):
            x = x + x.at[lanes ^ sft].get(mode="promise_in_bounds")
        return x

    racc = jnp.zeros((16,), jnp.float32)
    for ps in range(NPASS):

        def issue(g, c):
            gb = g * 16
            vu = iu[ps, pl.ds(gb, 16)]
            vp = ip_[ps, pl.ds(gb, 16)]
            vn = in_[ps, pl.ds(gb, 16)]
            for r in range(16):
                pltpu.async_copy(table.at[vu[r]], ru.at[gb + r], sem)
                pltpu.async_copy(table.at[vp[r]], rp.at[gb + r], sem)
                pltpu.async_copy(table.at[vn[r]], rn.at[gb + r], sem)
            return c

        lax.fori_loop(0, PCH // 16, issue, 0)

        def drain(k, c):
            pltpu.make_async_copy(table.at[0], ru.at[0], sem).wait()
            pltpu.make_async_copy(table.at[0], rp.at[0], sem).wait()
            pltpu.make_async_copy(table.at[0], rn.at[0], sem).wait()
            return c

        lax.fori_loop(0, PCH, drain, 0)

        def group(g, racc):
            rbase = g * 16
            dv = jnp.zeros((16,), jnp.float32)
            for r in range(16):
                row = rbase + r
                acc = jnp.zeros((16,), jnp.float32)
                for k in range(D // 16):
                    sl = pl.ds(k * 16, 16)
                    u = ru[row, sl]
                    p = rp[row, sl]
                    n = rn[row, sl]
                    su = u - jnp.minimum(jnp.maximum(u, nt), t)
                    sp = p - jnp.minimum(jnp.maximum(p, nt), t)
                    sn = n - jnp.minimum(jnp.maximum(n, nt), t)
                    acc = acc + su * (sn - sp)
                    racc = racc + (u * u + p * p + n * n)
                dv = jnp.where(lanes == r, lane_sum(acc), dv)
            dbuf[pl.ds(ps * PCH + rbase, 16)] = dv
            return racc

        racc = lax.fori_loop(0, PCH // 16, group, racc)

    rstage[...] = racc
    pltpu.sync_copy(dbuf, diffs_out.at[pl.ds(wid * BPW, BPW)])
    pltpu.sync_copy(rstage, regs_out.at[wid])


def _tc_body(d_ref, r_ref, s_ref, loss_ref, le_ref, reg_ref, sl_ref):
    diff = d_ref[...]
    le = jnp.mean(jax.nn.softplus(diff))
    reg = 0.5 * jnp.sum(r_ref[...]) * (1.0 / B) * REG_W
    sv = s_ref[0]
    sl = 0.5 * sv * sv * (1.0 / B) * SREG_W
    le_ref[0] = le
    reg_ref[0] = reg
    sl_ref[0] = sl
    loss_ref[0] = le + reg + sl


_tc_final = pl.pallas_call(
    _tc_body,
    out_shape=[jax.ShapeDtypeStruct((1,), jnp.float32)] * 4,
    in_specs=[
        pl.BlockSpec(memory_space=pltpu.VMEM),
        pl.BlockSpec(memory_space=pltpu.VMEM),
        pl.BlockSpec(memory_space=pltpu.SMEM),
    ],
    out_specs=[pl.BlockSpec(memory_space=pltpu.SMEM)] * 4,
)


def kernel(users, pos, neg, table, s):
    s = s.astype(jnp.float32)
    u2 = users.astype(jnp.int32).reshape(B // CH, CH)
    p2 = pos.astype(jnp.int32).reshape(B // CH, CH)
    n2 = neg.astype(jnp.int32).reshape(B // CH, CH)
    thr16 = jnp.broadcast_to(jax.nn.sigmoid(s), (16,))
    diffs, regs = _sc_gather(u2, p2, n2, thr16, table)
    loss, le, reg, sl = _tc_final(diffs.reshape(B // CH, CH), regs, s)
    return loss[0], le[0], reg[0], sl[0]
